# baseline (speedup 1.0000x reference)
import functools

import jax
import jax.numpy as jnp
from jax import lax
from jax.experimental import pallas as pl
from jax.experimental.pallas import tpu as pltpu
from jax.experimental.pallas import tpu_sc as plsc

N = 10000
E = 160000
D = 256
H = 128
G = 128
NS = 16
NC = 2

CHUNK = 120
NBUF = 3
PAIRS = 7
GROUP = 42
CH = 84
EDGES_PER_TILE = CH * CHUNK
E_PAD = NS * EDGES_PER_TILE
NGROUPS = CH // GROUP
NROWS = N + 8

RPT = 624
RPT_LAST = 640
ROW0_LAST = 15 * RPT

_mesh = plsc.VectorSubcoreMesh(core_axis_name="c", subcore_axis_name="s")


@functools.partial(
    pl.kernel,
    out_type=jax.ShapeDtypeStruct((NC, N, H), jnp.float32),
    mesh=_mesh,
    scratch_types=[
        pltpu.VMEM((PAIRS, 2, 2, CHUNK), jnp.int32),
        pltpu.VMEM((NBUF, CHUNK, H), jnp.float32),
        pltpu.VMEM_SHARED((NROWS, H), jnp.float32),
        pltpu.SemaphoreType.DMA((NBUF,)),
        pltpu.SemaphoreType.DMA((NBUF,)),
        pltpu.SemaphoreType.DMA((PAIRS,)),
        pltpu.SemaphoreType.DMA,
    ],
)
def _sc_message(h2, edge_p, pre2, idxv, rows, acc, gsem, ssem, isem,
                initsem):
    c = lax.axis_index("c")
    s = lax.axis_index("s")
    table = h2.at[c]

    @pl.when(s < 15)
    def _():
        row0 = pl.multiple_of(s * RPT, 8)
        pltpu.async_copy(table.at[pl.ds(row0, RPT)],
                         acc.at[pl.ds(row0, RPT)], initsem)

    @pl.when(s == 15)
    def _():
        pltpu.async_copy(table.at[pl.ds(ROW0_LAST, RPT_LAST)],
                         acc.at[pl.ds(ROW0_LAST, RPT_LAST)], initsem)

    def fire_pair(q, r):
        pltpu.async_copy(edge_p.at[s, pl.ds(2 * q, 2)], idxv.at[r],
                         isem.at[r])

    def wait_pair(r):
        pltpu.make_async_copy(edge_p.at[0, pl.ds(0, 2)], idxv.at[r],
                              isem.at[r]).wait()

    def gidx(b):
        return ((b // 2) % PAIRS, b % 2)

    def fire_gather(b, br):
        r, k = gidx(b)
        pltpu.async_copy(table.at[idxv.at[r, k, 0]], rows.at[br],
                         gsem.at[br])

    def wait_gather(b, br):
        r, k = gidx(b)
        pltpu.make_async_copy(table.at[idxv.at[r, k, 0]], rows.at[br],
                              gsem.at[br]).wait()

    def fire_scatter(b, br):
        r, k = gidx(b)
        pltpu.async_copy(rows.at[br], acc.at[idxv.at[r, k, 1]],
                         ssem.at[br], add=True)

    def wait_scatter(b, br):
        r, k = gidx(b)
        pltpu.make_async_copy(rows.at[br], acc.at[idxv.at[r, k, 1]],
                              ssem.at[br]).wait()

    for q in range(PAIRS - 1):
        fire_pair(q, q)
    wait_pair(0)
    fire_gather(0, 0)
    fire_gather(1, 1)

    @pl.when(s < 15)
    def _():
        pltpu.make_async_copy(table.at[pl.ds(0, RPT)],
                              acc.at[pl.ds(0, RPT)], initsem).wait()

    @pl.when(s == 15)
    def _():
        pltpu.make_async_copy(table.at[pl.ds(0, RPT_LAST)],
                              acc.at[pl.ds(0, RPT_LAST)], initsem).wait()
    plsc.subcore_barrier()

    @pl.loop(0, NGROUPS)
    def _(g):
        for b in range(GROUP):
            i = g * GROUP + b

            @pl.when(i >= 1)
            def _():
                wait_scatter((b - 1) % GROUP, (b - 1) % NBUF)

            if b % 2 == 1:
                @pl.when(i + 11 < CH)
                def _():
                    fire_pair((i + 11) // 2, ((b + 11) // 2) % PAIRS)

            wait_gather(b, b % NBUF)
            fire_scatter(b, b % NBUF)

            @pl.when(i + 2 < CH)
            def _():
                if b % 2 == 0:
                    wait_pair(((b + 2) // 2) % PAIRS)
                fire_gather((b + 2) % GROUP, (b + 2) % NBUF)

    wait_scatter((CH - 1) % GROUP, (CH - 1) % NBUF)
    plsc.subcore_barrier()

    @pl.when(s < 15)
    def _():
        row0 = pl.multiple_of(s * RPT, 8)
        pltpu.sync_copy(acc.at[pl.ds(row0, RPT)],
                        pre2.at[c, pl.ds(row0, RPT)])

    @pl.when(s == 15)
    def _():
        pltpu.sync_copy(acc.at[pl.ds(ROW0_LAST, RPT_LAST)],
                        pre2.at[c, pl.ds(ROW0_LAST, RPT_LAST)])


BN = 2000
NB = N // BN


def _tc_mlp_body(pre_ref, w1_ref, b1_ref, w2_ref, b2_ref, out_ref):
    x = jnp.concatenate([pre_ref[0], pre_ref[1]], axis=1)
    t = jnp.maximum(
        jnp.dot(x, w1_ref[...], preferred_element_type=jnp.float32)
        + b1_ref[...], 0.0)
    y = jnp.maximum(
        jnp.dot(t, w2_ref[...], preferred_element_type=jnp.float32)
        + b2_ref[...], 0.0)
    out_ref[0] = y[:, :H]
    out_ref[1] = y[:, H:]


def _tc_mlp(pre2, w1, b1, w2, b2):
    return pl.pallas_call(
        _tc_mlp_body,
        grid=(NB,),
        in_specs=[
            pl.BlockSpec((NC, BN, H), lambda i: (0, i, 0)),
            pl.BlockSpec((D, D), lambda i: (0, 0)),
            pl.BlockSpec((1, D), lambda i: (0, 0)),
            pl.BlockSpec((D, D), lambda i: (0, 0)),
            pl.BlockSpec((1, D), lambda i: (0, 0)),
        ],
        out_specs=pl.BlockSpec((NC, BN, H), lambda i: (0, i, 0)),
        out_shape=jax.ShapeDtypeStruct((NC, N, H), jnp.float32),
    )(pre2, w1, b1, w2, b2)


def _tc_mlp_pool_body(pre_ref, w1_ref, b1_ref, w2_ref, b2_ref, batch_ref,
                      out_ref, cnt_ref):
    i = pl.program_id(0)
    x = jnp.concatenate([pre_ref[0], pre_ref[1]], axis=1)
    t = jnp.maximum(
        jnp.dot(x, w1_ref[...], preferred_element_type=jnp.float32)
        + b1_ref[...], 0.0)
    y = jnp.maximum(
        jnp.dot(t, w2_ref[...], preferred_element_type=jnp.float32)
        + b2_ref[...], 0.0)
    gi = lax.broadcasted_iota(jnp.int32, (G, 1), 0)
    onehot = (batch_ref[0] == gi).astype(jnp.float32)
    psum = jnp.dot(onehot, y, preferred_element_type=jnp.float32)
    pcnt = jnp.sum(onehot, axis=1, keepdims=True)

    @pl.when(i == 0)
    def _():
        out_ref[...] = psum
        cnt_ref[...] = pcnt

    @pl.when(i > 0)
    def _():
        out_ref[...] += psum
        cnt_ref[...] += pcnt

    @pl.when(i == NB - 1)
    def _():
        out_ref[...] = out_ref[...] / jnp.maximum(cnt_ref[...], 1.0)


def _tc_mlp_pool(pre2, w1, b1, w2, b2, batch_3d):
    return pl.pallas_call(
        _tc_mlp_pool_body,
        grid=(NB,),
        in_specs=[
            pl.BlockSpec((NC, BN, H), lambda i: (0, i, 0)),
            pl.BlockSpec((D, D), lambda i: (0, 0)),
            pl.BlockSpec((1, D), lambda i: (0, 0)),
            pl.BlockSpec((D, D), lambda i: (0, 0)),
            pl.BlockSpec((1, D), lambda i: (0, 0)),
            pl.BlockSpec((1, 1, BN), lambda i: (i, 0, 0)),
        ],
        out_specs=pl.BlockSpec((G, D), lambda i: (0, 0)),
        out_shape=jax.ShapeDtypeStruct((G, D), jnp.float32),
        scratch_shapes=[pltpu.VMEM((G, 1), jnp.float32)],
    )(pre2, w1, b1, w2, b2, batch_3d)


def kernel(x, edge_index, batch, W1_0, b1_0, W2_0, b2_0, W1_1, b1_1, W2_1,
           b2_1, W1_2, b1_2, W2_2, b2_2):
    pad = E_PAD - E
    pad_vals = jnp.stack([jnp.zeros((pad,), jnp.int32),
                          jnp.full((pad,), N, jnp.int32)])
    edge_p = jnp.concatenate([edge_index, pad_vals], axis=1)
    edge_p = edge_p.reshape(2, NS, CH, CHUNK).transpose(1, 2, 0, 3)

    batch_3d = batch.reshape(NB, 1, BN)

    h2 = jnp.stack([x[:, :H], x[:, H:]])
    weights = [(W1_0, b1_0, W2_0, b2_0), (W1_1, b1_1, W2_1, b2_1)]

    for (w1, b1, w2, b2) in weights:
        pre2 = _sc_message(h2, edge_p)
        h2 = _tc_mlp(pre2, w1, b1.reshape(1, D), w2, b2.reshape(1, D))

    pre2 = _sc_message(h2, edge_p)
    return _tc_mlp_pool(pre2, W1_2, b1_2.reshape(1, D), W2_2,
                        b2_2.reshape(1, D), batch_3d)
